# baseline (speedup 1.0000x reference)
import functools

import jax
import jax.numpy as jnp
import numpy as np
from jax import lax
from jax.experimental import pallas as pl
from jax.experimental.pallas import tpu as pltpu, tpu_sc as plsc

D_MODEL = 256
N_FRAMES = 3
N_LEVELS = 4
N_POINTS = 4
MH = 64
DH = 4
SPATIAL = ((64, 64), (32, 32), (16, 16), (8, 8))
LSI = (0, 4096, 5120, 5376)
S_FRAME = 5440
LEN_IN = S_FRAME * N_FRAMES
LEN_PAD = 16384
N_B = 2
LQ = 2048
NM = N_B * MH

TILE_V = 1024
TQ = 128
QC = 512
N_WORKERS = 32


def _value_proj_body(x_ref, w_ref, b_ref, o_ref):
    v = jnp.dot(x_ref[0], w_ref[...], preferred_element_type=jnp.float32, precision=lax.Precision.HIGHEST) + b_ref[...]
    bf = v.T.astype(jnp.bfloat16)
    lo = lax.bitcast_convert_type(bf[:128], jnp.uint16).astype(jnp.int32)
    hi = lax.bitcast_convert_type(bf[128:], jnp.uint16).astype(jnp.int32)
    o_ref[...] = (lo | (hi << 16)).reshape(MH, DH // 2, TILE_V)


def _value_proj(x, W, b):
    grid = (N_B, LEN_PAD // TILE_V)
    return pl.pallas_call(
        _value_proj_body,
        grid=grid,
        in_specs=[
            pl.BlockSpec((1, TILE_V, D_MODEL), lambda n, i: (n, i, 0)),
            pl.BlockSpec((D_MODEL, D_MODEL), lambda n, i: (0, 0)),
            pl.BlockSpec((D_MODEL,), lambda n, i: (0,)),
        ],
        out_specs=pl.BlockSpec((MH, DH // 2, TILE_V), lambda n, i: (n, 0, i)),
        out_shape=jax.ShapeDtypeStruct((NM, DH // 2, LEN_PAD), jnp.int32),
    )(x, W, b)


def _payload_body(q_ref, rp_ref, wo_ref, bo_ref, wa_ref, ba_ref, sc_ref,
                  gs_ref, ge_ref, xyz_ref, att_ref):
    q = q_ref[0]
    off = jnp.dot(q, wo_ref[...], preferred_element_type=jnp.float32, precision=lax.Precision.HIGHEST) + bo_ref[...]
    refb = jnp.dot(rp_ref[0], sc_ref[...], preferred_element_type=jnp.float32, precision=lax.Precision.HIGHEST)
    xyz = refb + off
    logits = jnp.dot(q, wa_ref[...], preferred_element_type=jnp.float32, precision=lax.Precision.HIGHEST) + ba_ref[...]
    e = jnp.exp(logits)
    s = jnp.dot(e, gs_ref[...], preferred_element_type=jnp.float32, precision=lax.Precision.HIGHEST)
    sm = e * jnp.dot(1.0 / s, ge_ref[...], preferred_element_type=jnp.float32, precision=lax.Precision.HIGHEST)
    xyz_ref[...] = xyz.T.reshape(1, MH, N_LEVELS * N_POINTS * 3, TQ)
    att_ref[...] = sm.T.reshape(1, MH, N_LEVELS * N_POINTS, TQ)


def _payload(query, rp12, W_off, b_off2, W_attn, b_attn, scale_mat, gsum, gexp):
    grid = (N_B, LQ // TQ)
    LP = N_LEVELS * N_POINTS
    return pl.pallas_call(
        _payload_body,
        grid=grid,
        in_specs=[
            pl.BlockSpec((1, TQ, D_MODEL), lambda n, i: (n, i, 0)),
            pl.BlockSpec((1, TQ, 12), lambda n, i: (n, i, 0)),
            pl.BlockSpec((D_MODEL, 3072), lambda n, i: (0, 0)),
            pl.BlockSpec((3072,), lambda n, i: (0,)),
            pl.BlockSpec((D_MODEL, 1024), lambda n, i: (0, 0)),
            pl.BlockSpec((1024,), lambda n, i: (0,)),
            pl.BlockSpec((12, 3072), lambda n, i: (0, 0)),
            pl.BlockSpec((1024, MH), lambda n, i: (0, 0)),
            pl.BlockSpec((MH, 1024), lambda n, i: (0, 0)),
        ],
        out_specs=[
            pl.BlockSpec((1, MH, LP * 3, TQ), lambda n, i: (n, 0, 0, i)),
            pl.BlockSpec((1, MH, LP, TQ), lambda n, i: (n, 0, 0, i)),
        ],
        out_shape=[
            jax.ShapeDtypeStruct((N_B, MH, LP * 3, LQ), jnp.float32),
            jax.ShapeDtypeStruct((N_B, MH, LP, LQ), jnp.float32),
        ],
    )(query, rp12, W_off, b_off2, W_attn, b_attn, scale_mat, gsum, gexp)


def _floor16(v):
    vi = v.astype(jnp.int32)
    vf = vi.astype(jnp.float32)
    neg = v < vf
    return jnp.where(neg, vi - 1, vi), jnp.where(neg, vf - 1.0, vf)


def _sc_sample(value_t, xyz, att, lvlc):
    mesh = plsc.VectorSubcoreMesh(core_axis_name="c", subcore_axis_name="s")
    LP = N_LEVELS * N_POINTS
    n_pairs = NM // N_WORKERS

    @functools.partial(
        pl.kernel,
        out_type=jax.ShapeDtypeStruct((NM, DH, LQ), jnp.float32),
        mesh=mesh,
        compiler_params=pltpu.CompilerParams(needs_layout_passes=False),
        scratch_types=[
            pltpu.VMEM(((DH // 2) * LEN_PAD,), jnp.int32),
            pltpu.VMEM((LP * 3, QC), jnp.float32),
            pltpu.VMEM((LP, QC), jnp.float32),
            pltpu.VMEM((DH, LQ), jnp.float32),
            pltpu.VMEM((LP * 5, 16), jnp.int32),
        ],
    )
    def body(value_hbm, xyz_hbm, att_hbm, lvlc_hbm, out_hbm,
             table, xyzv, attv, outv, lvlv):
        wid = lax.axis_index("s") * 2 + lax.axis_index("c")
        pltpu.sync_copy(lvlc_hbm, lvlv)

        def lp_body(args, acc):
            lp, qo = args
            a0, a1, a2, a3, b0, b1, b2, b3 = acc
            Wi = lvlv[5 * lp + 0]
            Wm1 = lvlv[5 * lp + 1]
            Hm1 = lvlv[5 * lp + 2]
            basev = lvlv[5 * lp + 3]
            x = xyzv[3 * lp + 0, pl.ds(qo, 16)]
            y = xyzv[3 * lp + 1, pl.ds(qo, 16)]
            t = xyzv[3 * lp + 2, pl.ds(qo, 16)]
            a = attv[lp, pl.ds(qo, 16)]
            x0, x0f = _floor16(x)
            y0, y0f = _floor16(y)
            t0, t0f = _floor16(t)
            fx = x - x0f
            fy = y - y0f
            ft = t - t0f
            wx0 = jnp.where((x0 >= 0) & (x0 <= Wm1), 1.0 - fx, 0.0)
            wx1 = jnp.where((x0 >= -1) & (x0 < Wm1), fx, 0.0)
            wy0 = jnp.where((y0 >= 0) & (y0 <= Hm1), 1.0 - fy, 0.0)
            wy1 = jnp.where((y0 >= -1) & (y0 < Hm1), fy, 0.0)
            wt0 = jnp.where((t0 >= 0) & (t0 < N_FRAMES), 1.0 - ft, 0.0) * a
            wt1 = jnp.where((t0 >= -1) & (t0 < N_FRAMES - 1), ft, 0.0) * a
            zero = jnp.zeros((16,), jnp.int32)
            xc0 = jnp.minimum(jnp.maximum(x0, zero), Wm1)
            xc1 = jnp.minimum(jnp.maximum(x0 + 1, zero), Wm1)
            yc0 = jnp.minimum(jnp.maximum(y0, zero), Hm1) * Wi
            yc1 = jnp.minimum(jnp.maximum(y0 + 1, zero), Hm1) * Wi
            tc0 = jnp.clip(t0, 0, N_FRAMES - 1) * S_FRAME + basev
            tc1 = jnp.clip(t0 + 1, 0, N_FRAMES - 1) * S_FRAME + basev
            for ci, (r, wr) in enumerate(((tc0 + yc0, wt0 * wy0),
                                          (tc0 + yc1, wt0 * wy1),
                                          (tc1 + yc0, wt1 * wy0),
                                          (tc1 + yc1, wt1 * wy1))):
                for (xc, wx) in ((xc0, wx0), (xc1, wx1)):
                    idx = r + xc
                    w = wr * wx
                    gw0 = plsc.load_gather(table, [idx])
                    ve, vo = plsc.unpack(plsc.bitcast(gw0, jnp.bfloat16),
                                         format=plsc.PackFormat.INTERLEAVED)
                    gw1 = plsc.load_gather(table, [idx + LEN_PAD])
                    ve1, vo1 = plsc.unpack(plsc.bitcast(gw1, jnp.bfloat16),
                                           format=plsc.PackFormat.INTERLEAVED)
                    if ci < 2:
                        a0 = a0 + ve * w
                        a1 = a1 + vo * w
                        a2 = a2 + ve1 * w
                        a3 = a3 + vo1 * w
                    else:
                        b0 = b0 + ve * w
                        b1 = b1 + vo * w
                        b2 = b2 + ve1 * w
                        b3 = b3 + vo1 * w
            return (a0, a1, a2, a3, b0, b1, b2, b3)

        def pair_body(p, _):
            nm = wid * n_pairs + p
            pltpu.sync_copy(value_hbm.at[nm], table)

            def qc_body(qc, _):
                qb = pl.multiple_of(qc * QC, QC)
                pltpu.sync_copy(xyz_hbm.at[nm, :, pl.ds(qb, QC)], xyzv)
                pltpu.sync_copy(att_hbm.at[nm, :, pl.ds(qb, QC)], attv)

                def qv_body(qv, _):
                    qo = pl.multiple_of(qv * 16, 16)
                    z = jnp.zeros((16,), jnp.float32)
                    acc = plsc.parallel_loop(
                        0, LP, 1, unroll=2, carry=(z,) * 8)(
                            lambda lp, c: lp_body((lp, qo), c))
                    for dd in range(DH):
                        outv[dd, pl.ds(qb + qo, 16)] = acc[dd] + acc[dd + 4]
                    return 0

                lax.fori_loop(0, QC // 16, qv_body, 0)
                return 0

            lax.fori_loop(0, LQ // QC, qc_body, 0)
            pltpu.sync_copy(outv, out_hbm.at[nm])
            return 0

        lax.fori_loop(0, n_pairs, pair_body, 0)

    return body(value_t, xyz, att, lvlc)


def _out_proj_body(s_ref, w_ref, b_ref, o_ref):
    y = lax.dot_general(s_ref[0], w_ref[...], (((0,), (0,)), ((), ())),
                        preferred_element_type=jnp.float32, precision=lax.Precision.HIGHEST)
    o_ref[...] = (y + b_ref[...])[None]


def _out_proj(sc_out, W, b):
    return pl.pallas_call(
        _out_proj_body,
        grid=(N_B,),
        in_specs=[
            pl.BlockSpec((1, D_MODEL, LQ), lambda n: (n, 0, 0)),
            pl.BlockSpec((D_MODEL, D_MODEL), lambda n: (0, 0)),
            pl.BlockSpec((D_MODEL,), lambda n: (0,)),
        ],
        out_specs=pl.BlockSpec((1, LQ, D_MODEL), lambda n: (n, 0, 0)),
        out_shape=jax.ShapeDtypeStruct((N_B, LQ, D_MODEL), jnp.float32),
    )(sc_out, W, b)


def _consts():
    sc = np.zeros((12, 3072), np.float32)
    for m in range(MH):
        for lvl in range(N_LEVELS):
            H, W = SPATIAL[lvl]
            s3 = (W, H, N_FRAMES)
            for p in range(N_POINTS):
                for c in range(3):
                    sc[lvl * 3 + c, ((m * N_LEVELS + lvl) * N_POINTS + p) * 3 + c] = s3[c]
    gsum = np.zeros((1024, MH), np.float32)
    gexp = np.zeros((MH, 1024), np.float32)
    for i in range(1024):
        gsum[i, i // 16] = 1.0
        gexp[i // 16, i] = 1.0
    return jnp.asarray(sc), jnp.asarray(gsum), jnp.asarray(gexp)


def kernel(query, reference_points, input_flatten, input_spatial_shapes,
           input_level_start_index, W_value, b_value, W_offsets, b_offsets,
           W_attn, b_attn, W_out, b_out):
    scale_mat, gsum, gexp = _consts()
    perm = np.concatenate([np.arange(0, D_MODEL, 2), np.arange(1, D_MODEL, 2)])
    value_t = _value_proj(input_flatten, W_value[:, perm], b_value[perm])
    rp12 = reference_points.reshape(N_B, LQ, 12)
    xyz, att = _payload(query, rp12, W_offsets, b_offsets - 0.5, W_attn, b_attn,
                        scale_mat, gsum, gexp)
    lvlc = np.zeros((N_LEVELS * N_POINTS * 5, 16), np.int32)
    for lvl in range(N_LEVELS):
        H, W = SPATIAL[lvl]
        for pt in range(N_POINTS):
            lp = lvl * N_POINTS + pt
            lvlc[5 * lp + 0] = W
            lvlc[5 * lp + 1] = W - 1
            lvlc[5 * lp + 2] = H - 1
            lvlc[5 * lp + 3] = LSI[lvl]
    sc_out = _sc_sample(value_t.reshape(NM, (DH // 2) * LEN_PAD),
                        xyz.reshape(NM, N_LEVELS * N_POINTS * 3, LQ),
                        att.reshape(NM, N_LEVELS * N_POINTS, LQ),
                        jnp.asarray(lvlc))
    return _out_proj(sc_out.reshape(N_B, D_MODEL, LQ), W_out, b_out)


# speedup vs baseline: 1.3417x; 1.3417x over previous
import functools

import jax
import jax.numpy as jnp
import numpy as np
from jax import lax
from jax.experimental import pallas as pl
from jax.experimental.pallas import tpu as pltpu, tpu_sc as plsc

D_MODEL = 256
N_FRAMES = 3
N_LEVELS = 4
N_POINTS = 4
MH = 64
DH = 4
LP = N_LEVELS * N_POINTS
SPATIAL = ((64, 64), (32, 32), (16, 16), (8, 8))
LSI = (0, 4096, 5120, 5376)
S_FRAME = 5440
LEN_IN = S_FRAME * N_FRAMES
LEN_PAD = 16384
N_B = 2
LQ = 2048
NM = N_B * MH

TILE_V = 1024
TQ = 128
QC = 512
N_WORKERS = 32


def _value_proj_body(x_ref, w_ref, b_ref, o0_ref, o1_ref):
    v = jnp.dot(x_ref[0], w_ref[...], preferred_element_type=jnp.float32,
                precision=lax.Precision.HIGHEST) + b_ref[...]
    bf = v.T.astype(jnp.bfloat16)
    lo = lax.bitcast_convert_type(bf[:128], jnp.uint16).astype(jnp.int32)
    hi = lax.bitcast_convert_type(bf[128:], jnp.uint16).astype(jnp.int32)
    word = lo | (hi << 16)
    o0_ref[...] = word[:64]
    o1_ref[...] = word[64:]


def _value_proj(x, W, b):
    grid = (N_B, LEN_PAD // TILE_V)
    return pl.pallas_call(
        _value_proj_body,
        grid=grid,
        in_specs=[
            pl.BlockSpec((1, TILE_V, D_MODEL), lambda n, i: (n, i, 0)),
            pl.BlockSpec((D_MODEL, D_MODEL), lambda n, i: (0, 0)),
            pl.BlockSpec((D_MODEL,), lambda n, i: (0,)),
        ],
        out_specs=[
            pl.BlockSpec((MH, TILE_V), lambda n, i: (n, i)),
            pl.BlockSpec((MH, TILE_V), lambda n, i: (n, i)),
        ],
        out_shape=[
            jax.ShapeDtypeStruct((NM, LEN_PAD), jnp.int32),
            jax.ShapeDtypeStruct((NM, LEN_PAD), jnp.int32),
        ],
    )(x, W, b)


def _packbf(a, b):
    al = lax.bitcast_convert_type(a.astype(jnp.bfloat16), jnp.uint16).astype(jnp.int32)
    bl = lax.bitcast_convert_type(b.astype(jnp.bfloat16), jnp.uint16).astype(jnp.int32)
    return al | (bl << 16)


def _corner(u, um1):
    u0f = jnp.floor(u)
    fu = u - u0f
    u1f = u0f + 1.0
    w0 = jnp.where((u0f >= 0.0) & (u0f <= um1), 1.0 - fu, 0.0)
    w1 = jnp.where((u1f >= 0.0) & (u1f <= um1), fu, 0.0)
    c0 = jnp.clip(u0f, 0.0, um1).astype(jnp.int32)
    c1 = jnp.clip(u1f, 0.0, um1).astype(jnp.int32)
    return w0, w1, c0, c1


def _payload_body(q_ref, rp_ref, wo_ref, bo_ref, wa_ref, ba_ref, sc_ref,
                  gs_ref, ge_ref, cwm1_ref, chm1_ref, cwi_ref, cb_ref,
                  wx_ref, wy_ref, wt_ref, ix_ref, iy_ref, it_ref):
    hp = lax.Precision.HIGHEST
    q = q_ref[0]
    off = jnp.dot(q, wo_ref[...], preferred_element_type=jnp.float32, precision=hp) + bo_ref[...]
    refb = jnp.dot(rp_ref[0], sc_ref[...], preferred_element_type=jnp.float32, precision=hp)
    xyz = refb + off
    x = xyz[:, 0:1024]
    y = xyz[:, 1024:2048]
    t = xyz[:, 2048:3072]
    logits = jnp.dot(q, wa_ref[...], preferred_element_type=jnp.float32, precision=hp) + ba_ref[...]
    e = jnp.exp(logits)
    s = jnp.dot(e, gs_ref[...], preferred_element_type=jnp.float32, precision=hp)
    sm = e * jnp.dot(1.0 / s, ge_ref[...], preferred_element_type=jnp.float32, precision=hp)

    wx0, wx1, xc0, xc1 = _corner(x, cwm1_ref[...])
    wy0, wy1, yc0, yc1 = _corner(y, chm1_ref[...])
    wt0, wt1, tc0, tc1 = _corner(t, float(N_FRAMES - 1))
    wt0 = wt0 * sm
    wt1 = wt1 * sm
    Wi = cwi_ref[...]
    bs = cb_ref[...]
    ixw = xc0 | (xc1 << 16)
    iyw = (yc0 * Wi) | ((yc1 * Wi) << 16)
    itw = (tc0 * S_FRAME + bs) | ((tc1 * S_FRAME + bs) << 16)
    for ref, arr in ((wx_ref, _packbf(wx0, wx1)),
                     (wy_ref, _packbf(wy0, wy1)),
                     (wt_ref, _packbf(wt0, wt1)),
                     (ix_ref, ixw), (iy_ref, iyw), (it_ref, itw)):
        ref[...] = arr.T.reshape(1, MH, LP, TQ)


def _payload(query, rp12, W_off, b_off2, W_attn, b_attn, scale_mat, gsum, gexp,
             cwm1, chm1, cwi, cb):
    grid = (N_B, LQ // TQ)
    full = lambda n, i: (0, 0)
    out_spec = pl.BlockSpec((1, MH, LP, TQ), lambda n, i: (n, 0, 0, i))
    out_shape = jax.ShapeDtypeStruct((N_B, MH, LP, LQ), jnp.int32)
    return pl.pallas_call(
        _payload_body,
        grid=grid,
        in_specs=[
            pl.BlockSpec((1, TQ, D_MODEL), lambda n, i: (n, i, 0)),
            pl.BlockSpec((1, TQ, 12), lambda n, i: (n, i, 0)),
            pl.BlockSpec((D_MODEL, 3072), full),
            pl.BlockSpec((3072,), lambda n, i: (0,)),
            pl.BlockSpec((D_MODEL, 1024), full),
            pl.BlockSpec((1024,), lambda n, i: (0,)),
            pl.BlockSpec((12, 3072), full),
            pl.BlockSpec((1024, MH), full),
            pl.BlockSpec((MH, 1024), full),
            pl.BlockSpec((1, 1024), full),
            pl.BlockSpec((1, 1024), full),
            pl.BlockSpec((1, 1024), full),
            pl.BlockSpec((1, 1024), full),
        ],
        out_specs=[out_spec] * 6,
        out_shape=[out_shape] * 6,
    )(query, rp12, W_off, b_off2, W_attn, b_attn, scale_mat, gsum, gexp,
      cwm1, chm1, cwi, cb)


def _sc_sample(v0, v1, pwx, pwy, pwt, pix, piy, pit):
    mesh = plsc.VectorSubcoreMesh(core_axis_name="c", subcore_axis_name="s")
    n_pairs = NM // N_WORKERS

    @functools.partial(
        pl.kernel,
        out_type=jax.ShapeDtypeStruct((NM, DH, LQ), jnp.float32),
        mesh=mesh,
        compiler_params=pltpu.CompilerParams(needs_layout_passes=False),
        scratch_types=[
            pltpu.VMEM((LEN_PAD,), jnp.int32),
            pltpu.VMEM((LEN_PAD,), jnp.int32),
            pltpu.VMEM((LP, QC), jnp.int32),
            pltpu.VMEM((LP, QC), jnp.int32),
            pltpu.VMEM((LP, QC), jnp.int32),
            pltpu.VMEM((LP, QC), jnp.int32),
            pltpu.VMEM((LP, QC), jnp.int32),
            pltpu.VMEM((LP, QC), jnp.int32),
            pltpu.VMEM((DH, LQ), jnp.float32),
        ],
    )
    def body(v0_hbm, v1_hbm, pwx_hbm, pwy_hbm, pwt_hbm, pix_hbm, piy_hbm, pit_hbm,
             out_hbm, t0v, t1v, vwx, vwy, vwt, vix, viy, vit, outv):
        wid = lax.axis_index("s") * 2 + lax.axis_index("c")

        def lp_body(lp, qo, acc):
            a0, a1, a2, a3 = acc
            wx0, wx1 = plsc.unpack(
                plsc.bitcast(vwx[lp, pl.ds(qo, 16)], jnp.bfloat16),
                format=plsc.PackFormat.INTERLEAVED)
            wy0, wy1 = plsc.unpack(
                plsc.bitcast(vwy[lp, pl.ds(qo, 16)], jnp.bfloat16),
                format=plsc.PackFormat.INTERLEAVED)
            wt0, wt1 = plsc.unpack(
                plsc.bitcast(vwt[lp, pl.ds(qo, 16)], jnp.bfloat16),
                format=plsc.PackFormat.INTERLEAVED)
            ixw = vix[lp, pl.ds(qo, 16)]
            iyw = viy[lp, pl.ds(qo, 16)]
            itw = vit[lp, pl.ds(qo, 16)]
            mask = jnp.int32(0xFFFF)
            xc0 = ixw & mask
            xc1 = lax.shift_right_logical(ixw, 16)
            yw0 = iyw & mask
            yw1 = lax.shift_right_logical(iyw, 16)
            tS0 = itw & mask
            tS1 = lax.shift_right_logical(itw, 16)
            for (r, wr) in ((tS0 + yw0, wt0 * wy0),
                            (tS0 + yw1, wt0 * wy1),
                            (tS1 + yw0, wt1 * wy0),
                            (tS1 + yw1, wt1 * wy1)):
                for (xc, wx) in ((xc0, wx0), (xc1, wx1)):
                    idx = r + xc
                    w = wr * wx
                    ve, vo = plsc.unpack(
                        plsc.bitcast(plsc.load_gather(t0v, [idx]), jnp.bfloat16),
                        format=plsc.PackFormat.INTERLEAVED)
                    a0 = a0 + ve * w
                    a1 = a1 + vo * w
                    ve, vo = plsc.unpack(
                        plsc.bitcast(plsc.load_gather(t1v, [idx]), jnp.bfloat16),
                        format=plsc.PackFormat.INTERLEAVED)
                    a2 = a2 + ve * w
                    a3 = a3 + vo * w
            return (a0, a1, a2, a3)

        def pair_body(p, _):
            nm = wid * n_pairs + p
            pltpu.sync_copy(v0_hbm.at[nm], t0v)
            pltpu.sync_copy(v1_hbm.at[nm], t1v)

            def qc_body(qc, _):
                qb = pl.multiple_of(qc * QC, QC)
                pltpu.sync_copy(pwx_hbm.at[nm, :, pl.ds(qb, QC)], vwx)
                pltpu.sync_copy(pwy_hbm.at[nm, :, pl.ds(qb, QC)], vwy)
                pltpu.sync_copy(pwt_hbm.at[nm, :, pl.ds(qb, QC)], vwt)
                pltpu.sync_copy(pix_hbm.at[nm, :, pl.ds(qb, QC)], vix)
                pltpu.sync_copy(piy_hbm.at[nm, :, pl.ds(qb, QC)], viy)
                pltpu.sync_copy(pit_hbm.at[nm, :, pl.ds(qb, QC)], vit)

                def qv_body(qv, _):
                    qo = pl.multiple_of(qv * 16, 16)
                    z = jnp.zeros((16,), jnp.float32)
                    acc = lax.fori_loop(
                        0, LP, lambda lp, c: lp_body(lp, qo, c), (z, z, z, z))
                    for dd in range(DH):
                        outv[dd, pl.ds(qb + qo, 16)] = acc[dd]
                    return 0

                lax.fori_loop(0, QC // 16, qv_body, 0)
                return 0

            lax.fori_loop(0, LQ // QC, qc_body, 0)
            pltpu.sync_copy(outv, out_hbm.at[nm])
            return 0

        lax.fori_loop(0, n_pairs, pair_body, 0)

    return body(v0, v1, pwx, pwy, pwt, pix, piy, pit)


def _out_proj_body(s_ref, w_ref, b_ref, o_ref):
    y = lax.dot_general(s_ref[0], w_ref[...], (((0,), (0,)), ((), ())),
                        preferred_element_type=jnp.float32, precision=lax.Precision.HIGHEST)
    o_ref[...] = (y + b_ref[...])[None]


def _out_proj(sc_out, W, b):
    return pl.pallas_call(
        _out_proj_body,
        grid=(N_B,),
        in_specs=[
            pl.BlockSpec((1, D_MODEL, LQ), lambda n: (n, 0, 0)),
            pl.BlockSpec((D_MODEL, D_MODEL), lambda n: (0, 0)),
            pl.BlockSpec((D_MODEL,), lambda n: (0,)),
        ],
        out_specs=pl.BlockSpec((1, LQ, D_MODEL), lambda n: (n, 0, 0)),
        out_shape=jax.ShapeDtypeStruct((N_B, LQ, D_MODEL), jnp.float32),
    )(sc_out, W, b)


def _consts():
    perm_off = np.zeros(3072, np.int64)
    for c in range(3):
        for m in range(MH):
            for lvl in range(N_LEVELS):
                for p in range(N_POINTS):
                    col = ((m * N_LEVELS + lvl) * N_POINTS + p) * 3 + c
                    perm_off[c * 1024 + (m * N_LEVELS + lvl) * N_POINTS + p] = col
    sc = np.zeros((12, 3072), np.float32)
    for c in range(3):
        for m in range(MH):
            for lvl in range(N_LEVELS):
                H, W = SPATIAL[lvl]
                s3 = (W, H, N_FRAMES)
                for p in range(N_POINTS):
                    sc[lvl * 3 + c, c * 1024 + (m * N_LEVELS + lvl) * N_POINTS + p] = s3[c]
    gsum = np.zeros((1024, MH), np.float32)
    gexp = np.zeros((MH, 1024), np.float32)
    for i in range(1024):
        gsum[i, i // 16] = 1.0
        gexp[i // 16, i] = 1.0
    wm1 = np.zeros((1, 1024), np.float32)
    hm1 = np.zeros((1, 1024), np.float32)
    wi = np.zeros((1, 1024), np.int32)
    bs = np.zeros((1, 1024), np.int32)
    for m in range(MH):
        for lvl in range(N_LEVELS):
            H, W = SPATIAL[lvl]
            for p in range(N_POINTS):
                j = (m * N_LEVELS + lvl) * N_POINTS + p
                wm1[0, j] = W - 1
                hm1[0, j] = H - 1
                wi[0, j] = W
                bs[0, j] = LSI[lvl]
    perm_v = np.zeros(256, np.int64)
    for j in range(128):
        perm_v[j] = 4 * (j % 64) + 2 * (j // 64)
        perm_v[128 + j] = perm_v[j] + 1
    return (perm_off, jnp.asarray(sc), jnp.asarray(gsum), jnp.asarray(gexp),
            jnp.asarray(wm1), jnp.asarray(hm1), jnp.asarray(wi), jnp.asarray(bs),
            perm_v)


def kernel(query, reference_points, input_flatten, input_spatial_shapes,
           input_level_start_index, W_value, b_value, W_offsets, b_offsets,
           W_attn, b_attn, W_out, b_out):
    perm_off, scale_mat, gsum, gexp, wm1, hm1, wi, bs, perm_v = _consts()
    v0, v1 = _value_proj(input_flatten, W_value[:, perm_v], b_value[perm_v])
    rp12 = reference_points.reshape(N_B, LQ, 12)
    pay = _payload(query, rp12, W_offsets[:, perm_off], b_offsets[perm_off] - 0.5,
                   W_attn, b_attn, scale_mat, gsum, gexp, wm1, hm1, wi, bs)
    pay = [p.reshape(NM, LP, LQ) for p in pay]
    sc_out = _sc_sample(v0, v1, *pay)
    return _out_proj(sc_out.reshape(N_B, D_MODEL, LQ), W_out, b_out)


# speedup vs baseline: 1.4004x; 1.0438x over previous
import functools

import jax
import jax.numpy as jnp
import numpy as np
from jax import lax
from jax.experimental import pallas as pl
from jax.experimental.pallas import tpu as pltpu, tpu_sc as plsc

D_MODEL = 256
N_FRAMES = 3
N_LEVELS = 4
N_POINTS = 4
MH = 64
DH = 4
LP = N_LEVELS * N_POINTS
SPATIAL = ((64, 64), (32, 32), (16, 16), (8, 8))
LSI = (0, 4096, 5120, 5376)
S_FRAME = 5440
LEN_IN = S_FRAME * N_FRAMES
LEN_PAD = 16384
N_B = 2
LQ = 2048
NM = N_B * MH

TILE_V = 1024
TQ = 128
QC = 512
N_WORKERS = 32


def _value_proj_body(x_ref, w_ref, b_ref, o0_ref, o1_ref):
    v = jnp.dot(x_ref[0], w_ref[...], preferred_element_type=jnp.float32,
                precision=lax.Precision.HIGHEST) + b_ref[...]
    bf = v.T.astype(jnp.bfloat16)
    lo = lax.bitcast_convert_type(bf[:128], jnp.uint16).astype(jnp.int32)
    hi = lax.bitcast_convert_type(bf[128:], jnp.uint16).astype(jnp.int32)
    word = lo | (hi << 16)
    o0_ref[...] = word[:64]
    o1_ref[...] = word[64:]


def _value_proj(x, W, b):
    grid = (N_B, LEN_PAD // TILE_V)
    return pl.pallas_call(
        _value_proj_body,
        grid=grid,
        in_specs=[
            pl.BlockSpec((1, TILE_V, D_MODEL), lambda n, i: (n, i, 0)),
            pl.BlockSpec((D_MODEL, D_MODEL), lambda n, i: (0, 0)),
            pl.BlockSpec((D_MODEL,), lambda n, i: (0,)),
        ],
        out_specs=[
            pl.BlockSpec((MH, TILE_V), lambda n, i: (n, i)),
            pl.BlockSpec((MH, TILE_V), lambda n, i: (n, i)),
        ],
        out_shape=[
            jax.ShapeDtypeStruct((NM, LEN_PAD), jnp.int32),
            jax.ShapeDtypeStruct((NM, LEN_PAD), jnp.int32),
        ],
    )(x, W, b)


def _packbf(a, b):
    al = lax.bitcast_convert_type(a.astype(jnp.bfloat16), jnp.uint16).astype(jnp.int32)
    bl = lax.bitcast_convert_type(b.astype(jnp.bfloat16), jnp.uint16).astype(jnp.int32)
    return al | (bl << 16)


def _corner(u, um1):
    u0f = jnp.floor(u)
    fu = u - u0f
    u1f = u0f + 1.0
    w0 = jnp.where((u0f >= 0.0) & (u0f <= um1), 1.0 - fu, 0.0)
    w1 = jnp.where((u1f >= 0.0) & (u1f <= um1), fu, 0.0)
    c0 = jnp.clip(u0f, 0.0, um1).astype(jnp.int32)
    c1 = jnp.clip(u1f, 0.0, um1).astype(jnp.int32)
    return w0, w1, c0, c1


def _payload_body(q_ref, rp_ref, wo_ref, bo_ref, wa_ref, ba_ref, sc_ref,
                  gs_ref, ge_ref, cwm1_ref, chm1_ref, cwi_ref, cb_ref,
                  wx_ref, wy_ref, wt_ref, ix_ref, iy_ref, it_ref):
    hp = lax.Precision.HIGHEST
    qT = q_ref[0]
    offT = jnp.dot(wo_ref[...], qT, preferred_element_type=jnp.float32) + bo_ref[...]
    refbT = jnp.dot(sc_ref[...], rp_ref[0], preferred_element_type=jnp.float32, precision=hp)
    xyz = refbT + offT
    x = xyz[0:1024]
    y = xyz[1024:2048]
    t = xyz[2048:3072]
    logits = jnp.dot(wa_ref[...], qT, preferred_element_type=jnp.float32) + ba_ref[...]
    e = jnp.exp(logits)
    s = jnp.dot(gs_ref[...], e, preferred_element_type=jnp.float32, precision=hp)
    sm = e * jnp.dot(ge_ref[...], 1.0 / s, preferred_element_type=jnp.float32, precision=hp)

    wx0, wx1, xc0, xc1 = _corner(x, cwm1_ref[...])
    wy0, wy1, yc0, yc1 = _corner(y, chm1_ref[...])
    wt0, wt1, tc0, tc1 = _corner(t, float(N_FRAMES - 1))
    wt0 = wt0 * sm
    wt1 = wt1 * sm
    Wi = cwi_ref[...]
    bs = cb_ref[...]
    ixw = xc0 | (xc1 << 16)
    iyw = (yc0 * Wi) | ((yc1 * Wi) << 16)
    itw = (tc0 * S_FRAME + bs) | ((tc1 * S_FRAME + bs) << 16)
    for ref, arr in ((wx_ref, _packbf(wx0, wx1)),
                     (wy_ref, _packbf(wy0, wy1)),
                     (wt_ref, _packbf(wt0, wt1)),
                     (ix_ref, ixw), (iy_ref, iyw), (it_ref, itw)):
        ref[...] = arr.reshape(1, MH, LP, TQ)


def _payload(queryT, rp12T, W_offT, b_off2, W_attnT, b_attnc, scale_matT, gsumT,
             gexpT, cwm1, chm1, cwi, cb):
    grid = (N_B, LQ // TQ)
    full = lambda n, i: (0, 0)
    out_spec = pl.BlockSpec((1, MH, LP, TQ), lambda n, i: (n, 0, 0, i))
    out_shape = jax.ShapeDtypeStruct((N_B, MH, LP, LQ), jnp.int32)
    return pl.pallas_call(
        _payload_body,
        grid=grid,
        in_specs=[
            pl.BlockSpec((1, D_MODEL, TQ), lambda n, i: (n, 0, i)),
            pl.BlockSpec((1, 12, TQ), lambda n, i: (n, 0, i)),
            pl.BlockSpec((3072, D_MODEL), full),
            pl.BlockSpec((3072, 1), full),
            pl.BlockSpec((1024, D_MODEL), full),
            pl.BlockSpec((1024, 1), full),
            pl.BlockSpec((3072, 12), full),
            pl.BlockSpec((MH, 1024), full),
            pl.BlockSpec((1024, MH), full),
            pl.BlockSpec((1024, 1), full),
            pl.BlockSpec((1024, 1), full),
            pl.BlockSpec((1024, 1), full),
            pl.BlockSpec((1024, 1), full),
        ],
        out_specs=[out_spec] * 6,
        out_shape=[out_shape] * 6,
    )(queryT, rp12T, W_offT, b_off2, W_attnT, b_attnc, scale_matT, gsumT, gexpT,
      cwm1, chm1, cwi, cb)


def _sc_sample(v0, v1, pwx, pwy, pwt, pix, piy, pit):
    mesh = plsc.VectorSubcoreMesh(core_axis_name="c", subcore_axis_name="s")
    n_pairs = NM // N_WORKERS

    @functools.partial(
        pl.kernel,
        out_type=jax.ShapeDtypeStruct((NM, DH, LQ), jnp.float32),
        mesh=mesh,
        compiler_params=pltpu.CompilerParams(needs_layout_passes=False),
        scratch_types=[
            pltpu.VMEM((LEN_PAD,), jnp.int32),
            pltpu.VMEM((LEN_PAD,), jnp.int32),
            pltpu.VMEM((LP, QC), jnp.int32),
            pltpu.VMEM((LP, QC), jnp.int32),
            pltpu.VMEM((LP, QC), jnp.int32),
            pltpu.VMEM((LP, QC), jnp.int32),
            pltpu.VMEM((LP, QC), jnp.int32),
            pltpu.VMEM((LP, QC), jnp.int32),
            pltpu.VMEM((DH, LQ), jnp.float32),
        ],
    )
    def body(v0_hbm, v1_hbm, pwx_hbm, pwy_hbm, pwt_hbm, pix_hbm, piy_hbm, pit_hbm,
             out_hbm, t0v, t1v, vwx, vwy, vwt, vix, viy, vit, outv):
        wid = lax.axis_index("s") * 2 + lax.axis_index("c")

        def lp_body(lp, qo, acc):
            a0, a1, a2, a3 = acc
            wx0, wx1 = plsc.unpack(
                plsc.bitcast(vwx[lp, pl.ds(qo, 16)], jnp.bfloat16),
                format=plsc.PackFormat.INTERLEAVED)
            wy0, wy1 = plsc.unpack(
                plsc.bitcast(vwy[lp, pl.ds(qo, 16)], jnp.bfloat16),
                format=plsc.PackFormat.INTERLEAVED)
            wt0, wt1 = plsc.unpack(
                plsc.bitcast(vwt[lp, pl.ds(qo, 16)], jnp.bfloat16),
                format=plsc.PackFormat.INTERLEAVED)
            ixw = vix[lp, pl.ds(qo, 16)]
            iyw = viy[lp, pl.ds(qo, 16)]
            itw = vit[lp, pl.ds(qo, 16)]
            mask = jnp.int32(0xFFFF)
            xc0 = ixw & mask
            xc1 = lax.shift_right_logical(ixw, 16)
            yw0 = iyw & mask
            yw1 = lax.shift_right_logical(iyw, 16)
            tS0 = itw & mask
            tS1 = lax.shift_right_logical(itw, 16)
            for (r, wr) in ((tS0 + yw0, wt0 * wy0),
                            (tS0 + yw1, wt0 * wy1),
                            (tS1 + yw0, wt1 * wy0),
                            (tS1 + yw1, wt1 * wy1)):
                for (xc, wx) in ((xc0, wx0), (xc1, wx1)):
                    idx = r + xc
                    w = wr * wx
                    ve, vo = plsc.unpack(
                        plsc.bitcast(plsc.load_gather(t0v, [idx]), jnp.bfloat16),
                        format=plsc.PackFormat.INTERLEAVED)
                    a0 = a0 + ve * w
                    a1 = a1 + vo * w
                    ve, vo = plsc.unpack(
                        plsc.bitcast(plsc.load_gather(t1v, [idx]), jnp.bfloat16),
                        format=plsc.PackFormat.INTERLEAVED)
                    a2 = a2 + ve * w
                    a3 = a3 + vo * w
            return (a0, a1, a2, a3)

        def pair_body(p, _):
            nm = wid * n_pairs + p
            pltpu.sync_copy(v0_hbm.at[nm], t0v)
            pltpu.sync_copy(v1_hbm.at[nm], t1v)

            def qc_body(qc, _):
                qb = pl.multiple_of(qc * QC, QC)
                pltpu.sync_copy(pwx_hbm.at[nm, :, pl.ds(qb, QC)], vwx)
                pltpu.sync_copy(pwy_hbm.at[nm, :, pl.ds(qb, QC)], vwy)
                pltpu.sync_copy(pwt_hbm.at[nm, :, pl.ds(qb, QC)], vwt)
                pltpu.sync_copy(pix_hbm.at[nm, :, pl.ds(qb, QC)], vix)
                pltpu.sync_copy(piy_hbm.at[nm, :, pl.ds(qb, QC)], viy)
                pltpu.sync_copy(pit_hbm.at[nm, :, pl.ds(qb, QC)], vit)

                def qv_body(qv, _):
                    qo = pl.multiple_of(qv * 16, 16)
                    z = jnp.zeros((16,), jnp.float32)
                    acc = lax.fori_loop(
                        0, LP, lambda lp, c: lp_body(lp, qo, c), (z, z, z, z))
                    for dd in range(DH):
                        outv[dd, pl.ds(qb + qo, 16)] = acc[dd]
                    return 0

                lax.fori_loop(0, QC // 16, qv_body, 0)
                return 0

            lax.fori_loop(0, LQ // QC, qc_body, 0)
            pltpu.sync_copy(outv, out_hbm.at[nm])
            return 0

        lax.fori_loop(0, n_pairs, pair_body, 0)

    return body(v0, v1, pwx, pwy, pwt, pix, piy, pit)


def _out_proj_body(s_ref, w_ref, b_ref, o_ref):
    y = lax.dot_general(s_ref[0], w_ref[...], (((0,), (0,)), ((), ())),
                        preferred_element_type=jnp.float32, precision=lax.Precision.HIGHEST)
    o_ref[...] = (y + b_ref[...])[None]


def _out_proj(sc_out, W, b):
    return pl.pallas_call(
        _out_proj_body,
        grid=(N_B,),
        in_specs=[
            pl.BlockSpec((1, D_MODEL, LQ), lambda n: (n, 0, 0)),
            pl.BlockSpec((D_MODEL, D_MODEL), lambda n: (0, 0)),
            pl.BlockSpec((D_MODEL,), lambda n: (0,)),
        ],
        out_specs=pl.BlockSpec((1, LQ, D_MODEL), lambda n: (n, 0, 0)),
        out_shape=jax.ShapeDtypeStruct((N_B, LQ, D_MODEL), jnp.float32),
    )(sc_out, W, b)


def _consts():
    perm_off = np.zeros(3072, np.int64)
    for c in range(3):
        for m in range(MH):
            for lvl in range(N_LEVELS):
                for p in range(N_POINTS):
                    col = ((m * N_LEVELS + lvl) * N_POINTS + p) * 3 + c
                    perm_off[c * 1024 + (m * N_LEVELS + lvl) * N_POINTS + p] = col
    sc = np.zeros((12, 3072), np.float32)
    for c in range(3):
        for m in range(MH):
            for lvl in range(N_LEVELS):
                H, W = SPATIAL[lvl]
                s3 = (W, H, N_FRAMES)
                for p in range(N_POINTS):
                    sc[lvl * 3 + c, c * 1024 + (m * N_LEVELS + lvl) * N_POINTS + p] = s3[c]
    gsumT = np.zeros((MH, 1024), np.float32)
    gexpT = np.zeros((1024, MH), np.float32)
    for i in range(1024):
        gsumT[i // 16, i] = 1.0
        gexpT[i, i // 16] = 1.0
    wm1 = np.zeros((1024, 1), np.float32)
    hm1 = np.zeros((1024, 1), np.float32)
    wi = np.zeros((1024, 1), np.int32)
    bs = np.zeros((1024, 1), np.int32)
    for m in range(MH):
        for lvl in range(N_LEVELS):
            H, W = SPATIAL[lvl]
            for p in range(N_POINTS):
                j = (m * N_LEVELS + lvl) * N_POINTS + p
                wm1[j, 0] = W - 1
                hm1[j, 0] = H - 1
                wi[j, 0] = W
                bs[j, 0] = LSI[lvl]
    perm_v = np.zeros(256, np.int64)
    for j in range(128):
        perm_v[j] = 4 * (j % 64) + 2 * (j // 64)
        perm_v[128 + j] = perm_v[j] + 1
    return (perm_off, jnp.asarray(sc.T.copy()), jnp.asarray(gsumT),
            jnp.asarray(gexpT), jnp.asarray(wm1), jnp.asarray(hm1),
            jnp.asarray(wi), jnp.asarray(bs), perm_v)


def kernel(query, reference_points, input_flatten, input_spatial_shapes,
           input_level_start_index, W_value, b_value, W_offsets, b_offsets,
           W_attn, b_attn, W_out, b_out):
    perm_off, scale_matT, gsumT, gexpT, wm1, hm1, wi, bs, perm_v = _consts()
    v0, v1 = _value_proj(input_flatten, W_value[:, perm_v], b_value[perm_v])
    queryT = jnp.transpose(query, (0, 2, 1)).astype(jnp.bfloat16)
    rp12T = jnp.transpose(reference_points.reshape(N_B, LQ, 12), (0, 2, 1))
    pay = _payload(queryT, rp12T,
                   W_offsets[:, perm_off].T.astype(jnp.bfloat16),
                   (b_offsets[perm_off] - 0.5).reshape(3072, 1),
                   W_attn.T.astype(jnp.bfloat16), b_attn.reshape(1024, 1),
                   scale_matT, gsumT, gexpT, wm1, hm1, wi, bs)
    pay = [p.reshape(NM, LP, LQ) for p in pay]
    sc_out = _sc_sample(v0, v1, *pay)
    return _out_proj(sc_out.reshape(N_B, D_MODEL, LQ), W_out, b_out)


# speedup vs baseline: 1.6329x; 1.1660x over previous
import functools

import jax
import jax.numpy as jnp
import numpy as np
from jax import lax
from jax.experimental import pallas as pl
from jax.experimental.pallas import tpu as pltpu, tpu_sc as plsc

D_MODEL = 256
N_FRAMES = 3
N_LEVELS = 4
N_POINTS = 4
MH = 64
DH = 4
LP = N_LEVELS * N_POINTS
SPATIAL = ((64, 64), (32, 32), (16, 16), (8, 8))
LSI = (0, 4096, 5120, 5376)
S_FRAME = 5440
LEN_IN = S_FRAME * N_FRAMES
LEN_PAD = 16384
N_B = 2
LQ = 2048
NM = N_B * MH

TILE_V = 1024
TQ = 128
QC = 256
N_WORKERS = 32


def _value_proj_body(x_ref, w_ref, b_ref, o0_ref, o1_ref):
    v = jnp.dot(x_ref[0], w_ref[...], preferred_element_type=jnp.float32,
                precision=lax.Precision.HIGHEST) + b_ref[...]
    bf = v.T.astype(jnp.bfloat16)
    lo = lax.bitcast_convert_type(bf[:128], jnp.uint16).astype(jnp.int32)
    hi = lax.bitcast_convert_type(bf[128:], jnp.uint16).astype(jnp.int32)
    word = lo | (hi << 16)
    o0_ref[...] = word[:64]
    o1_ref[...] = word[64:]


def _value_proj(x, W, b):
    grid = (N_B, LEN_PAD // TILE_V)
    return pl.pallas_call(
        _value_proj_body,
        grid=grid,
        in_specs=[
            pl.BlockSpec((1, TILE_V, D_MODEL), lambda n, i: (n, i, 0)),
            pl.BlockSpec((D_MODEL, D_MODEL), lambda n, i: (0, 0)),
            pl.BlockSpec((D_MODEL,), lambda n, i: (0,)),
        ],
        out_specs=[
            pl.BlockSpec((MH, TILE_V), lambda n, i: (n, i)),
            pl.BlockSpec((MH, TILE_V), lambda n, i: (n, i)),
        ],
        out_shape=[
            jax.ShapeDtypeStruct((NM, LEN_PAD), jnp.int32),
            jax.ShapeDtypeStruct((NM, LEN_PAD), jnp.int32),
        ],
    )(x, W, b)


def _packbf(a, b):
    al = lax.bitcast_convert_type(a.astype(jnp.bfloat16), jnp.uint16).astype(jnp.int32)
    bl = lax.bitcast_convert_type(b.astype(jnp.bfloat16), jnp.uint16).astype(jnp.int32)
    return al | (bl << 16)


def _corner(u, um1):
    u0f = jnp.floor(u)
    fu = u - u0f
    u1f = u0f + 1.0
    w0 = jnp.where((u0f >= 0.0) & (u0f <= um1), 1.0 - fu, 0.0)
    w1 = jnp.where((u1f >= 0.0) & (u1f <= um1), fu, 0.0)
    c0 = jnp.clip(u0f, 0.0, um1).astype(jnp.int32)
    c1 = jnp.clip(u1f, 0.0, um1).astype(jnp.int32)
    return w0, w1, c0, c1


def _payload_body(q_ref, rp_ref, wo_ref, bo_ref, wa_ref, ba_ref, sc_ref,
                  gs_ref, ge_ref, cwm1_ref, chm1_ref, cwi_ref, cb_ref,
                  wx_ref, wy_ref, wt_ref, ix_ref, iy_ref, it_ref):
    hp = lax.Precision.HIGHEST
    qT = q_ref[0]
    offT = jnp.dot(wo_ref[...], qT, preferred_element_type=jnp.float32) + bo_ref[...]
    refbT = jnp.dot(sc_ref[...], rp_ref[0], preferred_element_type=jnp.float32, precision=hp)
    xyz = refbT + offT
    x = xyz[0:1024]
    y = xyz[1024:2048]
    t = xyz[2048:3072]
    logits = jnp.dot(wa_ref[...], qT, preferred_element_type=jnp.float32) + ba_ref[...]
    e = jnp.exp(logits)
    s = jnp.dot(gs_ref[...], e, preferred_element_type=jnp.float32, precision=hp)
    sm = e * jnp.dot(ge_ref[...], 1.0 / s, preferred_element_type=jnp.float32, precision=hp)

    wx0, wx1, xc0, xc1 = _corner(x, cwm1_ref[...])
    wy0, wy1, yc0, yc1 = _corner(y, chm1_ref[...])
    wt0, wt1, tc0, tc1 = _corner(t, float(N_FRAMES - 1))
    wt0 = wt0 * sm
    wt1 = wt1 * sm
    Wi = cwi_ref[...]
    bs = cb_ref[...]
    ixw = xc0 | (xc1 << 16)
    iyw = (yc0 * Wi) | ((yc1 * Wi) << 16)
    itw = (tc0 * S_FRAME + bs) | ((tc1 * S_FRAME + bs) << 16)
    for ref, arr in ((wx_ref, _packbf(wx0, wx1)),
                     (wy_ref, _packbf(wy0, wy1)),
                     (wt_ref, _packbf(wt0, wt1)),
                     (ix_ref, ixw), (iy_ref, iyw), (it_ref, itw)):
        ref[...] = arr.reshape(1, MH, LP, TQ)


def _payload(queryT, rp12T, W_offT, b_off2, W_attnT, b_attnc, scale_matT, gsumT,
             gexpT, cwm1, chm1, cwi, cb):
    grid = (N_B, LQ // TQ)
    full = lambda n, i: (0, 0)
    out_spec = pl.BlockSpec((1, MH, LP, TQ), lambda n, i: (n, 0, 0, i))
    out_shape = jax.ShapeDtypeStruct((N_B, MH, LP, LQ), jnp.int32)
    return pl.pallas_call(
        _payload_body,
        grid=grid,
        in_specs=[
            pl.BlockSpec((1, D_MODEL, TQ), lambda n, i: (n, 0, i)),
            pl.BlockSpec((1, 12, TQ), lambda n, i: (n, 0, i)),
            pl.BlockSpec((3072, D_MODEL), full),
            pl.BlockSpec((3072, 1), full),
            pl.BlockSpec((1024, D_MODEL), full),
            pl.BlockSpec((1024, 1), full),
            pl.BlockSpec((3072, 12), full),
            pl.BlockSpec((MH, 1024), full),
            pl.BlockSpec((1024, MH), full),
            pl.BlockSpec((1024, 1), full),
            pl.BlockSpec((1024, 1), full),
            pl.BlockSpec((1024, 1), full),
            pl.BlockSpec((1024, 1), full),
        ],
        out_specs=[out_spec] * 6,
        out_shape=[out_shape] * 6,
    )(queryT, rp12T, W_offT, b_off2, W_attnT, b_attnc, scale_matT, gsumT, gexpT,
      cwm1, chm1, cwi, cb)


def _sc_sample(v0, v1, pwx, pwy, pwt, pix, piy, pit):
    mesh = plsc.VectorSubcoreMesh(core_axis_name="c", subcore_axis_name="s")
    n_pairs = NM // N_WORKERS

    @functools.partial(
        pl.kernel,
        out_type=jax.ShapeDtypeStruct((NM, DH, LQ), jnp.float32),
        mesh=mesh,
        compiler_params=pltpu.CompilerParams(needs_layout_passes=False),
        scratch_types=(
            [pltpu.VMEM((LEN_PAD,), jnp.int32)] * 2
            + [pltpu.VMEM((2, LP, QC), jnp.int32)] * 6
            + [pltpu.VMEM((DH, LQ), jnp.float32)]
            + [pltpu.SemaphoreType.DMA] * 2
        ),
    )
    def body(v0_hbm, v1_hbm, pwx_hbm, pwy_hbm, pwt_hbm, pix_hbm, piy_hbm, pit_hbm,
             out_hbm, t0v, t1v, bwx, bwy, bwt, bix, biy, bit, outv, sem0, sem1):
        wid = lax.axis_index("s") * 2 + lax.axis_index("c")
        planes = ((pwx_hbm, bwx), (pwy_hbm, bwy), (pwt_hbm, bwt),
                  (pix_hbm, bix), (piy_hbm, biy), (pit_hbm, bit))
        sems = (sem0, sem1)
        n_qc = LQ // QC

        def fire(nm, qc, b):
            qb = pl.multiple_of(qc * QC, QC)
            for (hbm, buf) in planes:
                pltpu.async_copy(hbm.at[nm, :, pl.ds(qb, QC)], buf.at[b], sems[b])

        def drain(nm, qc, b):
            qb = pl.multiple_of(qc * QC, QC)
            for (hbm, buf) in planes:
                pltpu.make_async_copy(hbm.at[nm, :, pl.ds(qb, QC)], buf.at[b],
                                      sems[b]).wait()

        def lp_body(lp, qo, acc, b):
            a0, a1, a2, a3 = acc
            wx0, wx1 = plsc.unpack(
                plsc.bitcast(bwx[b, lp, pl.ds(qo, 16)], jnp.bfloat16),
                format=plsc.PackFormat.INTERLEAVED)
            wy0, wy1 = plsc.unpack(
                plsc.bitcast(bwy[b, lp, pl.ds(qo, 16)], jnp.bfloat16),
                format=plsc.PackFormat.INTERLEAVED)
            wt0, wt1 = plsc.unpack(
                plsc.bitcast(bwt[b, lp, pl.ds(qo, 16)], jnp.bfloat16),
                format=plsc.PackFormat.INTERLEAVED)
            ixw = bix[b, lp, pl.ds(qo, 16)]
            iyw = biy[b, lp, pl.ds(qo, 16)]
            itw = bit[b, lp, pl.ds(qo, 16)]
            mask = jnp.int32(0xFFFF)
            xc0 = ixw & mask
            xc1 = lax.shift_right_logical(ixw, 16)
            yw0 = iyw & mask
            yw1 = lax.shift_right_logical(iyw, 16)
            tS0 = itw & mask
            tS1 = lax.shift_right_logical(itw, 16)
            for (r, wr) in ((tS0 + yw0, wt0 * wy0),
                            (tS0 + yw1, wt0 * wy1),
                            (tS1 + yw0, wt1 * wy0),
                            (tS1 + yw1, wt1 * wy1)):
                for (xc, wx) in ((xc0, wx0), (xc1, wx1)):
                    idx = r + xc
                    w = wr * wx
                    ve, vo = plsc.unpack(
                        plsc.bitcast(plsc.load_gather(t0v, [idx]), jnp.bfloat16),
                        format=plsc.PackFormat.INTERLEAVED)
                    a0 = a0 + ve * w
                    a1 = a1 + vo * w
                    ve, vo = plsc.unpack(
                        plsc.bitcast(plsc.load_gather(t1v, [idx]), jnp.bfloat16),
                        format=plsc.PackFormat.INTERLEAVED)
                    a2 = a2 + ve * w
                    a3 = a3 + vo * w
            return (a0, a1, a2, a3)

        def pair_body(p, _):
            nm = wid * n_pairs + p
            pltpu.sync_copy(v0_hbm.at[nm], t0v)
            pltpu.sync_copy(v1_hbm.at[nm], t1v)
            fire(nm, 0, 0)

            def qc2_body(qc2, _):
                for b in range(2):
                    qc = qc2 * 2 + b
                    qb = pl.multiple_of(qc * QC, QC)
                    drain(nm, qc, b)

                    @pl.when(qc + 1 < n_qc)
                    def _():
                        fire(nm, qc + 1, 1 - b)

                    def qv_body(qv, _):
                        qo = pl.multiple_of(qv * 16, 16)
                        z = jnp.zeros((16,), jnp.float32)
                        acc = lax.fori_loop(
                            0, LP, lambda lp, c: lp_body(lp, qo, c, b),
                            (z, z, z, z))
                        for dd in range(DH):
                            outv[dd, pl.ds(qb + qo, 16)] = acc[dd]
                        return 0

                    lax.fori_loop(0, QC // 16, qv_body, 0)
                return 0

            lax.fori_loop(0, n_qc // 2, qc2_body, 0)
            pltpu.sync_copy(outv, out_hbm.at[nm])
            return 0

        lax.fori_loop(0, n_pairs, pair_body, 0)

    return body(v0, v1, pwx, pwy, pwt, pix, piy, pit)


def _out_proj_body(s_ref, w_ref, b_ref, o_ref):
    y = lax.dot_general(s_ref[0], w_ref[...], (((0,), (0,)), ((), ())),
                        preferred_element_type=jnp.float32, precision=lax.Precision.HIGHEST)
    o_ref[...] = (y + b_ref[...])[None]


def _out_proj(sc_out, W, b):
    return pl.pallas_call(
        _out_proj_body,
        grid=(N_B,),
        in_specs=[
            pl.BlockSpec((1, D_MODEL, LQ), lambda n: (n, 0, 0)),
            pl.BlockSpec((D_MODEL, D_MODEL), lambda n: (0, 0)),
            pl.BlockSpec((D_MODEL,), lambda n: (0,)),
        ],
        out_specs=pl.BlockSpec((1, LQ, D_MODEL), lambda n: (n, 0, 0)),
        out_shape=jax.ShapeDtypeStruct((N_B, LQ, D_MODEL), jnp.float32),
    )(sc_out, W, b)


def _consts():
    perm_off = np.zeros(3072, np.int64)
    for c in range(3):
        for m in range(MH):
            for lvl in range(N_LEVELS):
                for p in range(N_POINTS):
                    col = ((m * N_LEVELS + lvl) * N_POINTS + p) * 3 + c
                    perm_off[c * 1024 + (m * N_LEVELS + lvl) * N_POINTS + p] = col
    sc = np.zeros((12, 3072), np.float32)
    for c in range(3):
        for m in range(MH):
            for lvl in range(N_LEVELS):
                H, W = SPATIAL[lvl]
                s3 = (W, H, N_FRAMES)
                for p in range(N_POINTS):
                    sc[lvl * 3 + c, c * 1024 + (m * N_LEVELS + lvl) * N_POINTS + p] = s3[c]
    gsumT = np.zeros((MH, 1024), np.float32)
    gexpT = np.zeros((1024, MH), np.float32)
    for i in range(1024):
        gsumT[i // 16, i] = 1.0
        gexpT[i, i // 16] = 1.0
    wm1 = np.zeros((1024, 1), np.float32)
    hm1 = np.zeros((1024, 1), np.float32)
    wi = np.zeros((1024, 1), np.int32)
    bs = np.zeros((1024, 1), np.int32)
    for m in range(MH):
        for lvl in range(N_LEVELS):
            H, W = SPATIAL[lvl]
            for p in range(N_POINTS):
                j = (m * N_LEVELS + lvl) * N_POINTS + p
                wm1[j, 0] = W - 1
                hm1[j, 0] = H - 1
                wi[j, 0] = W
                bs[j, 0] = LSI[lvl]
    perm_v = np.zeros(256, np.int64)
    for j in range(128):
        perm_v[j] = 4 * (j % 64) + 2 * (j // 64)
        perm_v[128 + j] = perm_v[j] + 1
    return (perm_off, jnp.asarray(sc.T.copy()), jnp.asarray(gsumT),
            jnp.asarray(gexpT), jnp.asarray(wm1), jnp.asarray(hm1),
            jnp.asarray(wi), jnp.asarray(bs), perm_v)


def kernel(query, reference_points, input_flatten, input_spatial_shapes,
           input_level_start_index, W_value, b_value, W_offsets, b_offsets,
           W_attn, b_attn, W_out, b_out):
    perm_off, scale_matT, gsumT, gexpT, wm1, hm1, wi, bs, perm_v = _consts()
    v0, v1 = _value_proj(input_flatten, W_value[:, perm_v], b_value[perm_v])
    queryT = jnp.transpose(query, (0, 2, 1)).astype(jnp.bfloat16)
    rp12T = jnp.transpose(reference_points.reshape(N_B, LQ, 12), (0, 2, 1))
    pay = _payload(queryT, rp12T,
                   W_offsets[:, perm_off].T.astype(jnp.bfloat16),
                   (b_offsets[perm_off] - 0.5).reshape(3072, 1),
                   W_attn.T.astype(jnp.bfloat16), b_attn.reshape(1024, 1),
                   scale_matT, gsumT, gexpT, wm1, hm1, wi, bs)
    pay = [p.reshape(NM, LP, LQ) for p in pay]
    sc_out = _sc_sample(v0, v1, *pay)
    return _out_proj(sc_out.reshape(N_B, D_MODEL, LQ), W_out, b_out)


# speedup vs baseline: 1.7030x; 1.0429x over previous
import functools

import jax
import jax.numpy as jnp
import numpy as np
from jax import lax
from jax.experimental import pallas as pl
from jax.experimental.pallas import tpu as pltpu, tpu_sc as plsc

D_MODEL = 256
N_FRAMES = 3
N_LEVELS = 4
N_POINTS = 4
MH = 64
DH = 4
LP = N_LEVELS * N_POINTS
SPATIAL = ((64, 64), (32, 32), (16, 16), (8, 8))
LSI = (0, 4096, 5120, 5376)
S_FRAME = 5440
LEN_IN = S_FRAME * N_FRAMES
LEN_PAD = 16384
N_B = 2
LQ = 2048
NM = N_B * MH

TILE_V = 1024
TQ = 256
QC = 256
N_WORKERS = 32


def _value_proj_body(x_ref, w_ref, b_ref, o0_ref, o1_ref):
    v = jnp.dot(x_ref[0].astype(jnp.bfloat16), w_ref[...],
                preferred_element_type=jnp.float32) + b_ref[...]
    bf = v.T.astype(jnp.bfloat16)
    lo = lax.bitcast_convert_type(bf[:128], jnp.uint16).astype(jnp.int32)
    hi = lax.bitcast_convert_type(bf[128:], jnp.uint16).astype(jnp.int32)
    word = lo | (hi << 16)
    o0_ref[...] = word[:64]
    o1_ref[...] = word[64:]


def _value_proj(x, W, b):
    grid = (N_B, LEN_PAD // TILE_V)
    return pl.pallas_call(
        _value_proj_body,
        grid=grid,
        in_specs=[
            pl.BlockSpec((1, TILE_V, D_MODEL), lambda n, i: (n, i, 0)),
            pl.BlockSpec((D_MODEL, D_MODEL), lambda n, i: (0, 0)),
            pl.BlockSpec((D_MODEL,), lambda n, i: (0,)),
        ],
        out_specs=[
            pl.BlockSpec((MH, TILE_V), lambda n, i: (n, i)),
            pl.BlockSpec((MH, TILE_V), lambda n, i: (n, i)),
        ],
        out_shape=[
            jax.ShapeDtypeStruct((NM, LEN_PAD), jnp.int32),
            jax.ShapeDtypeStruct((NM, LEN_PAD), jnp.int32),
        ],
    )(x, W, b)


def _packbf(a, b):
    al = lax.bitcast_convert_type(a.astype(jnp.bfloat16), jnp.uint16).astype(jnp.int32)
    bl = lax.bitcast_convert_type(b.astype(jnp.bfloat16), jnp.uint16).astype(jnp.int32)
    return al | (bl << 16)


def _corner(u, um1):
    u0f = jnp.floor(u)
    fu = u - u0f
    u1f = u0f + 1.0
    w0 = jnp.where((u0f >= 0.0) & (u0f <= um1), 1.0 - fu, 0.0)
    w1 = jnp.where((u1f >= 0.0) & (u1f <= um1), fu, 0.0)
    c0 = jnp.clip(u0f, 0.0, um1).astype(jnp.int32)
    c1 = jnp.clip(u1f, 0.0, um1).astype(jnp.int32)
    return w0, w1, c0, c1


def _payload_body(q_ref, rp_ref, wo_ref, bo_ref, wa_ref, ba_ref, sc_ref,
                  gs_ref, ge_ref, cwm1_ref, chm1_ref, cwi_ref, cb_ref,
                  wx_ref, wy_ref, wt_ref, ix_ref, iy_ref, it_ref):
    hp = lax.Precision.HIGHEST
    qT = q_ref[0]
    offT = jnp.dot(wo_ref[...], qT, preferred_element_type=jnp.float32) + bo_ref[...]
    refbT = jnp.dot(sc_ref[...], rp_ref[0], preferred_element_type=jnp.float32, precision=hp)
    xyz = refbT + offT
    x = xyz[0:1024]
    y = xyz[1024:2048]
    t = xyz[2048:3072]
    logits = jnp.dot(wa_ref[...], qT, preferred_element_type=jnp.float32) + ba_ref[...]
    e = jnp.exp(logits)
    s = jnp.dot(gs_ref[...], e, preferred_element_type=jnp.float32, precision=hp)
    sm = e * jnp.dot(ge_ref[...], 1.0 / s, preferred_element_type=jnp.float32, precision=hp)

    wx0, wx1, xc0, xc1 = _corner(x, cwm1_ref[...])
    wy0, wy1, yc0, yc1 = _corner(y, chm1_ref[...])
    wt0, wt1, tc0, tc1 = _corner(t, float(N_FRAMES - 1))
    wt0 = wt0 * sm
    wt1 = wt1 * sm
    Wi = cwi_ref[...]
    bs = cb_ref[...]
    ixw = xc0 | (xc1 << 16)
    iyw = (yc0 * Wi) | ((yc1 * Wi) << 16)
    itw = (tc0 * S_FRAME + bs) | ((tc1 * S_FRAME + bs) << 16)
    for ref, arr in ((wx_ref, _packbf(wx0, wx1)),
                     (wy_ref, _packbf(wy0, wy1)),
                     (wt_ref, _packbf(wt0, wt1)),
                     (ix_ref, ixw), (iy_ref, iyw), (it_ref, itw)):
        ref[...] = arr.reshape(1, MH, LP, TQ)


def _payload(queryT, rp12T, W_offT, b_off2, W_attnT, b_attnc, scale_matT, gsumT,
             gexpT, cwm1, chm1, cwi, cb):
    grid = (N_B, LQ // TQ)
    full = lambda n, i: (0, 0)
    out_spec = pl.BlockSpec((1, MH, LP, TQ), lambda n, i: (n, 0, 0, i))
    out_shape = jax.ShapeDtypeStruct((N_B, MH, LP, LQ), jnp.int32)
    return pl.pallas_call(
        _payload_body,
        grid=grid,
        in_specs=[
            pl.BlockSpec((1, D_MODEL, TQ), lambda n, i: (n, 0, i)),
            pl.BlockSpec((1, 12, TQ), lambda n, i: (n, 0, i)),
            pl.BlockSpec((3072, D_MODEL), full),
            pl.BlockSpec((3072, 1), full),
            pl.BlockSpec((1024, D_MODEL), full),
            pl.BlockSpec((1024, 1), full),
            pl.BlockSpec((3072, 12), full),
            pl.BlockSpec((MH, 1024), full),
            pl.BlockSpec((1024, MH), full),
            pl.BlockSpec((1024, 1), full),
            pl.BlockSpec((1024, 1), full),
            pl.BlockSpec((1024, 1), full),
            pl.BlockSpec((1024, 1), full),
        ],
        out_specs=[out_spec] * 6,
        out_shape=[out_shape] * 6,
    )(queryT, rp12T, W_offT, b_off2, W_attnT, b_attnc, scale_matT, gsumT, gexpT,
      cwm1, chm1, cwi, cb)


def _sc_sample(v0, v1, pwx, pwy, pwt, pix, piy, pit):
    mesh = plsc.VectorSubcoreMesh(core_axis_name="c", subcore_axis_name="s")
    n_pairs = NM // N_WORKERS

    @functools.partial(
        pl.kernel,
        out_type=jax.ShapeDtypeStruct((NM, DH, LQ), jnp.float32),
        mesh=mesh,
        compiler_params=pltpu.CompilerParams(needs_layout_passes=False),
        scratch_types=(
            [pltpu.VMEM((LEN_PAD,), jnp.int32)] * 2
            + [pltpu.VMEM((2, LP, QC), jnp.int32)] * 6
            + [pltpu.VMEM((DH, LQ), jnp.float32)]
            + [pltpu.SemaphoreType.DMA] * 2
        ),
    )
    def body(v0_hbm, v1_hbm, pwx_hbm, pwy_hbm, pwt_hbm, pix_hbm, piy_hbm, pit_hbm,
             out_hbm, t0v, t1v, bwx, bwy, bwt, bix, biy, bit, outv, sem0, sem1):
        wid = lax.axis_index("s") * 2 + lax.axis_index("c")
        planes = ((pwx_hbm, bwx), (pwy_hbm, bwy), (pwt_hbm, bwt),
                  (pix_hbm, bix), (piy_hbm, biy), (pit_hbm, bit))
        sems = (sem0, sem1)
        n_qc = LQ // QC

        def fire(nm, qc, b):
            qb = pl.multiple_of(qc * QC, QC)
            for (hbm, buf) in planes:
                pltpu.async_copy(hbm.at[nm, :, pl.ds(qb, QC)], buf.at[b], sems[b])

        def drain(nm, qc, b):
            qb = pl.multiple_of(qc * QC, QC)
            for (hbm, buf) in planes:
                pltpu.make_async_copy(hbm.at[nm, :, pl.ds(qb, QC)], buf.at[b],
                                      sems[b]).wait()

        def lp_body(lp, qo, acc, b):
            a0, a1, a2, a3 = acc
            wx0, wx1 = plsc.unpack(
                plsc.bitcast(bwx[b, lp, pl.ds(qo, 16)], jnp.bfloat16),
                format=plsc.PackFormat.INTERLEAVED)
            wy0, wy1 = plsc.unpack(
                plsc.bitcast(bwy[b, lp, pl.ds(qo, 16)], jnp.bfloat16),
                format=plsc.PackFormat.INTERLEAVED)
            wt0, wt1 = plsc.unpack(
                plsc.bitcast(bwt[b, lp, pl.ds(qo, 16)], jnp.bfloat16),
                format=plsc.PackFormat.INTERLEAVED)
            ixw = bix[b, lp, pl.ds(qo, 16)]
            iyw = biy[b, lp, pl.ds(qo, 16)]
            itw = bit[b, lp, pl.ds(qo, 16)]
            mask = jnp.int32(0xFFFF)
            xc0 = ixw & mask
            xc1 = lax.shift_right_logical(ixw, 16)
            yw0 = iyw & mask
            yw1 = lax.shift_right_logical(iyw, 16)
            tS0 = itw & mask
            tS1 = lax.shift_right_logical(itw, 16)
            for (r, wr) in ((tS0 + yw0, wt0 * wy0),
                            (tS0 + yw1, wt0 * wy1),
                            (tS1 + yw0, wt1 * wy0),
                            (tS1 + yw1, wt1 * wy1)):
                for (xc, wx) in ((xc0, wx0), (xc1, wx1)):
                    idx = r + xc
                    w = wr * wx
                    ve, vo = plsc.unpack(
                        plsc.bitcast(plsc.load_gather(t0v, [idx]), jnp.bfloat16),
                        format=plsc.PackFormat.INTERLEAVED)
                    a0 = a0 + ve * w
                    a1 = a1 + vo * w
                    ve, vo = plsc.unpack(
                        plsc.bitcast(plsc.load_gather(t1v, [idx]), jnp.bfloat16),
                        format=plsc.PackFormat.INTERLEAVED)
                    a2 = a2 + ve * w
                    a3 = a3 + vo * w
            return (a0, a1, a2, a3)

        def pair_body(p, _):
            nm = wid * n_pairs + p
            pltpu.sync_copy(v0_hbm.at[nm], t0v)
            pltpu.sync_copy(v1_hbm.at[nm], t1v)
            fire(nm, 0, 0)

            def qc2_body(qc2, _):
                for b in range(2):
                    qc = qc2 * 2 + b
                    qb = pl.multiple_of(qc * QC, QC)
                    drain(nm, qc, b)

                    @pl.when(qc + 1 < n_qc)
                    def _():
                        fire(nm, qc + 1, 1 - b)

                    def qv_body(qv, _):
                        qo = pl.multiple_of(qv * 16, 16)
                        z = jnp.zeros((16,), jnp.float32)
                        acc = lax.fori_loop(
                            0, LP, lambda lp, c: lp_body(lp, qo, c, b),
                            (z, z, z, z))
                        for dd in range(DH):
                            outv[dd, pl.ds(qb + qo, 16)] = acc[dd]
                        return 0

                    lax.fori_loop(0, QC // 16, qv_body, 0)
                return 0

            lax.fori_loop(0, n_qc // 2, qc2_body, 0)
            pltpu.sync_copy(outv, out_hbm.at[nm])
            return 0

        lax.fori_loop(0, n_pairs, pair_body, 0)

    return body(v0, v1, pwx, pwy, pwt, pix, piy, pit)


def _out_proj_body(s_ref, w_ref, b_ref, o_ref):
    y = lax.dot_general(s_ref[0], w_ref[...], (((0,), (0,)), ((), ())),
                        preferred_element_type=jnp.float32, precision=lax.Precision.HIGHEST)
    o_ref[...] = (y + b_ref[...])[None]


def _out_proj(sc_out, W, b):
    return pl.pallas_call(
        _out_proj_body,
        grid=(N_B,),
        in_specs=[
            pl.BlockSpec((1, D_MODEL, LQ), lambda n: (n, 0, 0)),
            pl.BlockSpec((D_MODEL, D_MODEL), lambda n: (0, 0)),
            pl.BlockSpec((D_MODEL,), lambda n: (0,)),
        ],
        out_specs=pl.BlockSpec((1, LQ, D_MODEL), lambda n: (n, 0, 0)),
        out_shape=jax.ShapeDtypeStruct((N_B, LQ, D_MODEL), jnp.float32),
    )(sc_out, W, b)


def _consts():
    perm_off = np.zeros(3072, np.int64)
    for c in range(3):
        for m in range(MH):
            for lvl in range(N_LEVELS):
                for p in range(N_POINTS):
                    col = ((m * N_LEVELS + lvl) * N_POINTS + p) * 3 + c
                    perm_off[c * 1024 + (m * N_LEVELS + lvl) * N_POINTS + p] = col
    sc = np.zeros((12, 3072), np.float32)
    for c in range(3):
        for m in range(MH):
            for lvl in range(N_LEVELS):
                H, W = SPATIAL[lvl]
                s3 = (W, H, N_FRAMES)
                for p in range(N_POINTS):
                    sc[lvl * 3 + c, c * 1024 + (m * N_LEVELS + lvl) * N_POINTS + p] = s3[c]
    gsumT = np.zeros((MH, 1024), np.float32)
    gexpT = np.zeros((1024, MH), np.float32)
    for i in range(1024):
        gsumT[i // 16, i] = 1.0
        gexpT[i, i // 16] = 1.0
    wm1 = np.zeros((1024, 1), np.float32)
    hm1 = np.zeros((1024, 1), np.float32)
    wi = np.zeros((1024, 1), np.int32)
    bs = np.zeros((1024, 1), np.int32)
    for m in range(MH):
        for lvl in range(N_LEVELS):
            H, W = SPATIAL[lvl]
            for p in range(N_POINTS):
                j = (m * N_LEVELS + lvl) * N_POINTS + p
                wm1[j, 0] = W - 1
                hm1[j, 0] = H - 1
                wi[j, 0] = W
                bs[j, 0] = LSI[lvl]
    perm_v = np.zeros(256, np.int64)
    for j in range(128):
        perm_v[j] = 4 * (j % 64) + 2 * (j // 64)
        perm_v[128 + j] = perm_v[j] + 1
    return (perm_off, jnp.asarray(sc.T.copy()), jnp.asarray(gsumT),
            jnp.asarray(gexpT), jnp.asarray(wm1), jnp.asarray(hm1),
            jnp.asarray(wi), jnp.asarray(bs), perm_v)


def kernel(query, reference_points, input_flatten, input_spatial_shapes,
           input_level_start_index, W_value, b_value, W_offsets, b_offsets,
           W_attn, b_attn, W_out, b_out):
    perm_off, scale_matT, gsumT, gexpT, wm1, hm1, wi, bs, perm_v = _consts()
    v0, v1 = _value_proj(input_flatten, W_value[:, perm_v].astype(jnp.bfloat16),
                         b_value[perm_v])
    queryT = jnp.transpose(query, (0, 2, 1)).astype(jnp.bfloat16)
    rp12T = jnp.transpose(reference_points.reshape(N_B, LQ, 12), (0, 2, 1))
    pay = _payload(queryT, rp12T,
                   W_offsets[:, perm_off].T.astype(jnp.bfloat16),
                   (b_offsets[perm_off] - 0.5).reshape(3072, 1),
                   W_attn.T.astype(jnp.bfloat16), b_attn.reshape(1024, 1),
                   scale_matT, gsumT, gexpT, wm1, hm1, wi, bs)
    pay = [p.reshape(NM, LP, LQ) for p in pay]
    sc_out = _sc_sample(v0, v1, *pay)
    return _out_proj(sc_out.reshape(N_B, D_MODEL, LQ), W_out, b_out)
